# baseline (speedup 1.0000x reference)
import jax
import jax.numpy as jnp
from jax.experimental import pallas as pl
from jax.experimental.pallas import tpu as pltpu

_LIST_DIM = [64, 64, 64, 64]
_E = 8
_K = 4
_OC = 128
_KS = 3
_BB = 8


def _cv2_row(v, n):
    mean = jnp.sum(v) / n
    var = jnp.sum((v - mean) ** 2) / (n - 1)
    return var / (mean * mean + 1e-10)


def _fused_kernel(x_ref, gl_ref, wg_ref, w1_ref, b1_ref, w2_ref, b2_ref,
                  out_ref, loss_ref, acc_ref):
    i = pl.program_id(0)
    n = pl.program_id(1)
    nb = pl.num_programs(1)
    L = x_ref.shape[2]
    Lp = L - _KS + 1
    BB = x_ref.shape[0]
    E = _E

    logits = jnp.dot(gl_ref[0], wg_ref[0],
                     preferred_element_type=jnp.float32)
    m = jnp.max(logits, axis=1, keepdims=True)
    ex = jnp.exp(logits - m)
    sm = ex / jnp.sum(ex, axis=1, keepdims=True)
    iota = jax.lax.broadcasted_iota(jnp.int32, (BB, E), 1)
    remaining = sm
    mask = jnp.zeros((BB, E), jnp.bool_)
    for _ in range(_K):
        rowmax = jnp.max(remaining, axis=1, keepdims=True)
        ismax = remaining == rowmax
        first = jnp.min(jnp.where(ismax, iota, E), axis=1, keepdims=True)
        sel = iota == first
        mask = jnp.logical_or(mask, sel)
        remaining = jnp.where(sel, -1.0, remaining)
    kept = jnp.where(mask, sm, 0.0)
    topsum = jnp.sum(kept, axis=1, keepdims=True)
    gates = kept / (topsum + 1e-6)

    imp = jnp.sum(gates, axis=0, keepdims=True)
    load = jnp.sum((gates > 0).astype(jnp.float32), axis=0, keepdims=True)

    @pl.when(jnp.logical_and(i == 0, n == 0))
    def _():
        loss_ref[:, :] = jnp.zeros((1, 1), jnp.float32)

    @pl.when(n == 0)
    def _():
        acc_ref[pl.ds(2 * i, 1), :E] = imp
        acc_ref[pl.ds(2 * i + 1, 1), :E] = load

    @pl.when(n == nb - 1)
    def _():
        imp_t = acc_ref[pl.ds(2 * i, 1), :E] + imp
        load_t = acc_ref[pl.ds(2 * i + 1, 1), :E] + load
        step_loss = (_cv2_row(imp_t, E) + _cv2_row(load_t, E)) * 0.01
        loss_ref[:, :] += jnp.reshape(step_loss, (1, 1))

    w2s = [w2_ref[0][:, e, :] for e in range(E)]
    b2blk = b2_ref[0]
    for bb in range(BB):
        x = x_ref[bb].astype(jnp.bfloat16)
        xcat = jnp.concatenate([x[:, k:k + Lp] for k in range(_KS)], axis=0)
        h = jnp.tanh(jnp.dot(w1_ref[0], xcat,
                             preferred_element_type=jnp.float32) + b1_ref[0])
        h = h.astype(jnp.bfloat16)
        weff = jnp.zeros((_OC, _OC), jnp.float32)
        beff = jnp.zeros((_OC, 1), jnp.float32)
        for e in range(E):
            g = gates[bb, e]
            weff = weff + g * w2s[e]
            beff = beff + g * b2blk[:, e:e + 1]
        out_ref[bb] = jnp.dot(weff.astype(jnp.bfloat16), h,
                              preferred_element_type=jnp.float32) + beff


def kernel(x, conv1_w, conv1_b, conv2_w, conv2_b, w_gate):
    B, D, L = x.shape
    S = len(_LIST_DIM)
    dim = _LIST_DIM[0]
    OC, E, KS = _OC, _E, _KS
    Lp = L - KS + 1
    NB = B // _BB

    gl = x[:, :, L - 6:L - 1].reshape(B, S, dim * 5).transpose(1, 0, 2)

    w1cat = jnp.transpose(conv1_w, (0, 1, 3, 2)).reshape(
        S, OC, KS * dim).astype(jnp.bfloat16)
    b1c = conv1_b[:, :, None]
    w2q = conv2_w[:, :, :, 0].reshape(S, OC, E, OC)
    b2q = conv2_b.reshape(S, OC, E)

    out, loss2d = pl.pallas_call(
        _fused_kernel,
        grid=(S, NB),
        in_specs=[
            pl.BlockSpec((_BB, dim, L), lambda i, n: (n, i, 0)),
            pl.BlockSpec((1, _BB, dim * 5), lambda i, n: (i, n, 0)),
            pl.BlockSpec((1, dim * 5, E), lambda i, n: (i, 0, 0)),
            pl.BlockSpec((1, OC, KS * dim), lambda i, n: (i, 0, 0)),
            pl.BlockSpec((1, OC, 1), lambda i, n: (i, 0, 0)),
            pl.BlockSpec((1, OC, E, OC), lambda i, n: (i, 0, 0, 0)),
            pl.BlockSpec((1, OC, E), lambda i, n: (i, 0, 0)),
        ],
        out_specs=[
            pl.BlockSpec((_BB, OC, Lp), lambda i, n: (n, i, 0)),
            pl.BlockSpec((1, 1), lambda i, n: (0, 0)),
        ],
        out_shape=[
            jax.ShapeDtypeStruct((B, S * OC, Lp), jnp.float32),
            jax.ShapeDtypeStruct((1, 1), jnp.float32),
        ],
        scratch_shapes=[pltpu.VMEM((8, 128), jnp.float32)],
        compiler_params=pltpu.CompilerParams(
            dimension_semantics=("arbitrary", "arbitrary")),
    )(x, gl, w_gate, w1cat, b1c, w2q, b2q)

    return out, loss2d[0, 0]


# speedup vs baseline: 1.2868x; 1.2868x over previous
import jax
import jax.numpy as jnp
from jax.experimental import pallas as pl
from jax.experimental.pallas import tpu as pltpu

_LIST_DIM = [64, 64, 64, 64]
_E = 8
_K = 4
_OC = 128
_KS = 3
_BB = 8


def _cv2_row(v, n):
    mean = jnp.sum(v) / n
    var = jnp.sum((v - mean) ** 2) / (n - 1)
    return var / (mean * mean + 1e-10)


def _fused_kernel(x_ref, gl_ref, wg_ref, w1_ref, b1_ref, w2_ref, b2_ref,
                  out_ref, loss_ref, acc_ref):
    i = pl.program_id(0)
    n = pl.program_id(1)
    nb = pl.num_programs(1)
    L = x_ref.shape[2]
    Lp = L - _KS + 1
    BB = x_ref.shape[0]
    E = _E

    logits = jnp.dot(gl_ref[0], wg_ref[0],
                     preferred_element_type=jnp.float32)
    m = jnp.max(logits, axis=1, keepdims=True)
    ex = jnp.exp(logits - m)
    sm = ex / jnp.sum(ex, axis=1, keepdims=True)
    iota = jax.lax.broadcasted_iota(jnp.int32, (BB, E), 1)
    remaining = sm
    mask = jnp.zeros((BB, E), jnp.bool_)
    for _ in range(_K):
        rowmax = jnp.max(remaining, axis=1, keepdims=True)
        ismax = remaining == rowmax
        first = jnp.min(jnp.where(ismax, iota, E), axis=1, keepdims=True)
        sel = iota == first
        mask = jnp.logical_or(mask, sel)
        remaining = jnp.where(sel, -1.0, remaining)
    kept = jnp.where(mask, sm, 0.0)
    topsum = jnp.sum(kept, axis=1, keepdims=True)
    gates = kept / (topsum + 1e-6)

    imp = jnp.sum(gates, axis=0, keepdims=True)
    load = jnp.sum((gates > 0).astype(jnp.float32), axis=0, keepdims=True)

    @pl.when(jnp.logical_and(i == 0, n == 0))
    def _():
        loss_ref[:, :] = jnp.zeros((1, 1), jnp.float32)

    @pl.when(n == 0)
    def _():
        acc_ref[pl.ds(2 * i, 1), :E] = imp
        acc_ref[pl.ds(2 * i + 1, 1), :E] = load

    @pl.when(n == nb - 1)
    def _():
        imp_t = acc_ref[pl.ds(2 * i, 1), :E] + imp
        load_t = acc_ref[pl.ds(2 * i + 1, 1), :E] + load
        step_loss = (_cv2_row(imp_t, E) + _cv2_row(load_t, E)) * 0.01
        loss_ref[:, :] += jnp.reshape(step_loss, (1, 1))

    w2s = [w2_ref[0, e] for e in range(E)]
    for bb in range(BB):
        x = x_ref[bb].astype(jnp.bfloat16)
        xcat = jnp.concatenate([x[:, k:k + Lp] for k in range(_KS)], axis=0)
        h = jnp.tanh(jnp.dot(w1_ref[0], xcat,
                             preferred_element_type=jnp.float32) + b1_ref[0])
        h = h.astype(jnp.bfloat16)
        weff = jnp.zeros((_OC, _OC), jnp.float32)
        beff = jnp.zeros((_OC, 1), jnp.float32)
        for e in range(E):
            g = gates[bb, e]
            weff = weff + g * w2s[e]
            beff = beff + g * b2_ref[0, e]
        out_ref[bb] = jnp.dot(weff.astype(jnp.bfloat16), h,
                              preferred_element_type=jnp.float32) + beff


def kernel(x, conv1_w, conv1_b, conv2_w, conv2_b, w_gate):
    B, D, L = x.shape
    S = len(_LIST_DIM)
    dim = _LIST_DIM[0]
    OC, E, KS = _OC, _E, _KS
    Lp = L - KS + 1
    NB = B // _BB

    gl = x[:, :, L - 6:L - 1].reshape(B, S, dim * 5).transpose(1, 0, 2)

    w1cat = jnp.transpose(conv1_w, (0, 1, 3, 2)).reshape(
        S, OC, KS * dim).astype(jnp.bfloat16)
    b1c = conv1_b[:, :, None]
    w2q = jnp.transpose(conv2_w[:, :, :, 0].reshape(S, OC, E, OC),
                        (0, 2, 1, 3))
    b2q = jnp.transpose(conv2_b.reshape(S, OC, E),
                        (0, 2, 1))[:, :, :, None]

    out, loss2d = pl.pallas_call(
        _fused_kernel,
        grid=(S, NB),
        in_specs=[
            pl.BlockSpec((_BB, dim, L), lambda i, n: (n, i, 0)),
            pl.BlockSpec((1, _BB, dim * 5), lambda i, n: (i, n, 0)),
            pl.BlockSpec((1, dim * 5, E), lambda i, n: (i, 0, 0)),
            pl.BlockSpec((1, OC, KS * dim), lambda i, n: (i, 0, 0)),
            pl.BlockSpec((1, OC, 1), lambda i, n: (i, 0, 0)),
            pl.BlockSpec((1, E, OC, OC), lambda i, n: (i, 0, 0, 0)),
            pl.BlockSpec((1, E, OC, 1), lambda i, n: (i, 0, 0, 0)),
        ],
        out_specs=[
            pl.BlockSpec((_BB, OC, Lp), lambda i, n: (n, i, 0)),
            pl.BlockSpec((1, 1), lambda i, n: (0, 0)),
        ],
        out_shape=[
            jax.ShapeDtypeStruct((B, S * OC, Lp), jnp.float32),
            jax.ShapeDtypeStruct((1, 1), jnp.float32),
        ],
        scratch_shapes=[pltpu.VMEM((8, 128), jnp.float32)],
        compiler_params=pltpu.CompilerParams(
            dimension_semantics=("arbitrary", "arbitrary")),
    )(x, gl, w_gate, w1cat, b1c, w2q, b2q)

    return out, loss2d[0, 0]
